# baseline (speedup 1.0000x reference)
import functools

import jax
import jax.numpy as jnp
from jax import lax
from jax.experimental import pallas as pl
from jax.experimental.pallas import tpu as pltpu
from jax.experimental.pallas import tpu_sc as plsc


def _sc_gather(idx3, x, nw, nch, k):
    total = nw * nch * k
    _, c = x.shape
    mesh = plsc.VectorSubcoreMesh(core_axis_name="c", subcore_axis_name="s")
    nc = mesh.num_cores

    @functools.partial(
        pl.kernel,
        out_type=jax.ShapeDtypeStruct((total, c), jnp.int32),
        mesh=mesh,
        scratch_types=[
            pltpu.VMEM((nch, k), jnp.int32),
            pltpu.VMEM((k, c), jnp.int32),
            pltpu.VMEM((k, c), jnp.int32),
            pltpu.SemaphoreType.DMA,
            pltpu.SemaphoreType.DMA,
        ],
        compiler_params=pltpu.CompilerParams(use_tc_tiling_on_sc=False),
    )
    def gather_kernel(idx_hbm, x_hbm, out_hbm, idx_v, bufa, bufb, sema, semb):
        wid = lax.axis_index("s") * nc + lax.axis_index("c")
        base = wid * (nch * k)
        pltpu.sync_copy(idx_hbm.at[wid], idx_v)

        def pair(j, carry):
            c0 = j * 2
            c1 = c0 + 1
            cpa = pltpu.async_copy(x_hbm.at[idx_v.at[c0]], bufa, sema)
            cpb = pltpu.async_copy(x_hbm.at[idx_v.at[c1]], bufb, semb)
            cpa.wait()
            pltpu.sync_copy(bufa, out_hbm.at[pl.ds(base + c0 * k, k)])
            cpb.wait()
            pltpu.sync_copy(bufb, out_hbm.at[pl.ds(base + c1 * k, k)])
            return carry

        lax.fori_loop(0, nch // 2, pair, 0)

    return gather_kernel(idx3, x)


def _mm_stats_body(x_ref, g_ref, w_ref, y_ref, s_ref):
    i = pl.program_id(0)
    g0 = g_ref[0]
    g1 = g_ref[1]
    g2 = g_ref[2]
    g3 = g_ref[3]
    feat = jnp.concatenate(
        [
            x_ref[...],
            jnp.minimum(g0, g1),
            jnp.maximum(g0, g1),
            jnp.minimum(g2, g3),
            jnp.maximum(g2, g3),
        ],
        axis=1,
    )
    y = jnp.dot(feat, w_ref[...], preferred_element_type=jnp.float32)
    y_ref[...] = y
    srow = jnp.sum(y, axis=0)[None]
    qrow = jnp.sum(y * y, axis=0)[None]
    blk = jnp.concatenate(
        [srow, qrow, jnp.zeros((6, y.shape[1]), jnp.float32)], axis=0
    )

    @pl.when(i == 0)
    def _():
        s_ref[...] = blk

    @pl.when(i != 0)
    def _():
        s_ref[...] += blk


def _norm_body(y_ref, s_ref, p_ref, o_ref, *, n_rows):
    inv_n = 1.0 / n_rows
    mean = s_ref[0] * inv_n
    var = s_ref[1] * inv_n - mean * mean
    inv = lax.rsqrt(var + 1e-5)
    scale = p_ref[0] * inv
    shift = p_ref[1] - mean * scale
    o_ref[...] = jnp.maximum(y_ref[...] * scale + shift, 0.0)


def kernel(x, nb, W, gamma, beta):
    e, c = x.shape
    c_out = W.shape[0]

    x_bf = x.astype(jnp.bfloat16)
    x_pack = lax.bitcast_convert_type(
        x_bf.reshape(e, c // 2, 2), jnp.int32
    )
    idx = jnp.clip(nb.astype(jnp.int32), 0, e - 1)
    idx_flat = idx.T.reshape(-1)

    nw = 32
    per_w = idx_flat.shape[0] // nw
    k = 80
    nch = per_w // k
    idx3 = idx_flat.reshape(nw, nch, k)

    g_pack = _sc_gather(idx3, x_pack, nw, nch, k)
    g = lax.bitcast_convert_type(g_pack, jnp.bfloat16).reshape(4, e, c)

    wt = W.T.astype(jnp.bfloat16)

    t = 1000
    grid = (e // t,)
    y, stats = pl.pallas_call(
        _mm_stats_body,
        grid=grid,
        in_specs=[
            pl.BlockSpec((t, c), lambda i: (i, 0)),
            pl.BlockSpec((4, t, c), lambda i: (0, i, 0)),
            pl.BlockSpec((5 * c, c_out), lambda i: (0, 0)),
        ],
        out_specs=[
            pl.BlockSpec((t, c_out), lambda i: (i, 0)),
            pl.BlockSpec((8, c_out), lambda i: (0, 0)),
        ],
        out_shape=[
            jax.ShapeDtypeStruct((e, c_out), jnp.float32),
            jax.ShapeDtypeStruct((8, c_out), jnp.float32),
        ],
    )(x_bf, g, wt)

    params = jnp.concatenate(
        [gamma[None], beta[None], jnp.zeros((6, c_out), jnp.float32)], axis=0
    )

    out = pl.pallas_call(
        functools.partial(_norm_body, n_rows=e),
        grid=grid,
        in_specs=[
            pl.BlockSpec((t, c_out), lambda i: (i, 0)),
            pl.BlockSpec((8, c_out), lambda i: (0, 0)),
            pl.BlockSpec((8, c_out), lambda i: (0, 0)),
        ],
        out_specs=pl.BlockSpec((t, c_out), lambda i: (i, 0)),
        out_shape=jax.ShapeDtypeStruct((e, c_out), jnp.float32),
    )(y, stats, params)

    return out


# speedup vs baseline: 5.0167x; 5.0167x over previous
import functools

import jax
import jax.numpy as jnp
from jax import lax
from jax.experimental import pallas as pl
from jax.experimental.pallas import tpu as pltpu
from jax.experimental.pallas import tpu_sc as plsc


def _sc_gather(idx3, x, nw, nch, k):
    total = nw * nch * k
    _, c = x.shape
    mesh = plsc.VectorSubcoreMesh(core_axis_name="c", subcore_axis_name="s")
    nc = mesh.num_cores

    @functools.partial(
        pl.kernel,
        out_type=jax.ShapeDtypeStruct((total, c), jnp.float32),
        mesh=mesh,
        scratch_types=[
            pltpu.VMEM((nch, k), jnp.int32),
            pltpu.VMEM((k, c), jnp.float32),
            pltpu.VMEM((k, c), jnp.float32),
            pltpu.SemaphoreType.DMA,
            pltpu.SemaphoreType.DMA,
        ],
    )
    def gather_kernel(idx_hbm, x_hbm, out_hbm, idx_v, bufa, bufb, sema, semb):
        wid = lax.axis_index("s") * nc + lax.axis_index("c")
        base = wid * (nch * k)
        pltpu.sync_copy(idx_hbm.at[wid], idx_v)

        def pair(j, carry):
            c0 = j * 2
            c1 = c0 + 1
            cpa = pltpu.async_copy(x_hbm.at[idx_v.at[c0]], bufa, sema)
            cpb = pltpu.async_copy(x_hbm.at[idx_v.at[c1]], bufb, semb)
            cpa.wait()
            pltpu.sync_copy(bufa, out_hbm.at[pl.ds(base + c0 * k, k)])
            cpb.wait()
            pltpu.sync_copy(bufb, out_hbm.at[pl.ds(base + c1 * k, k)])
            return carry

        lax.fori_loop(0, nch // 2, pair, 0)

    return gather_kernel(idx3, x)


def _mm_stats_body(x_ref, g_ref, w_ref, y_ref, s_ref):
    i = pl.program_id(0)
    g0 = g_ref[0]
    g1 = g_ref[1]
    g2 = g_ref[2]
    g3 = g_ref[3]
    feat = jnp.concatenate(
        [
            x_ref[...],
            jnp.minimum(g0, g1),
            jnp.maximum(g0, g1),
            jnp.minimum(g2, g3),
            jnp.maximum(g2, g3),
        ],
        axis=1,
    ).astype(jnp.bfloat16)
    y = jnp.dot(feat, w_ref[...], preferred_element_type=jnp.float32)
    y_ref[...] = y.astype(jnp.bfloat16)
    srow = jnp.sum(y, axis=0)[None]
    qrow = jnp.sum(y * y, axis=0)[None]
    blk = jnp.concatenate(
        [srow, qrow, jnp.zeros((6, y.shape[1]), jnp.float32)], axis=0
    )

    @pl.when(i == 0)
    def _():
        s_ref[...] = blk

    @pl.when(i != 0)
    def _():
        s_ref[...] += blk


def _norm_body(y_ref, s_ref, p_ref, o_ref, *, n_rows):
    inv_n = 1.0 / n_rows
    mean = s_ref[0] * inv_n
    var = s_ref[1] * inv_n - mean * mean
    inv = lax.rsqrt(var + 1e-5)
    scale = p_ref[0] * inv
    shift = p_ref[1] - mean * scale
    y = y_ref[...].astype(jnp.float32)
    o_ref[...] = jnp.maximum(y * scale + shift, 0.0)


def kernel(x, nb, W, gamma, beta):
    e, c = x.shape
    c_out = W.shape[0]

    idx = jnp.clip(nb.astype(jnp.int32), 0, e - 1)
    idx_flat = idx.T.reshape(-1)

    nw = 32
    per_w = idx_flat.shape[0] // nw
    k = 80
    nch = per_w // k
    idx3 = idx_flat.reshape(nw, nch, k)

    g_flat = _sc_gather(idx3, x, nw, nch, k)
    g = g_flat.reshape(4, e, c)

    wt = W.T.astype(jnp.bfloat16)

    t = 2000
    grid = (e // t,)
    y, stats = pl.pallas_call(
        _mm_stats_body,
        grid=grid,
        in_specs=[
            pl.BlockSpec((t, c), lambda i: (i, 0)),
            pl.BlockSpec((4, t, c), lambda i: (0, i, 0)),
            pl.BlockSpec((5 * c, c_out), lambda i: (0, 0)),
        ],
        out_specs=[
            pl.BlockSpec((t, c_out), lambda i: (i, 0)),
            pl.BlockSpec((8, c_out), lambda i: (0, 0)),
        ],
        out_shape=[
            jax.ShapeDtypeStruct((e, c_out), jnp.bfloat16),
            jax.ShapeDtypeStruct((8, c_out), jnp.float32),
        ],
    )(x, g, wt)

    params = jnp.concatenate(
        [gamma[None], beta[None], jnp.zeros((6, c_out), jnp.float32)], axis=0
    )

    out = pl.pallas_call(
        functools.partial(_norm_body, n_rows=e),
        grid=grid,
        in_specs=[
            pl.BlockSpec((t, c_out), lambda i: (i, 0)),
            pl.BlockSpec((8, c_out), lambda i: (0, 0)),
            pl.BlockSpec((8, c_out), lambda i: (0, 0)),
        ],
        out_specs=pl.BlockSpec((t, c_out), lambda i: (i, 0)),
        out_shape=jax.ShapeDtypeStruct((e, c_out), jnp.float32),
    )(y, stats, params)

    return out


# speedup vs baseline: 5.3982x; 1.0760x over previous
import functools

import jax
import jax.numpy as jnp
from jax import lax
from jax.experimental import pallas as pl
from jax.experimental.pallas import tpu as pltpu
from jax.experimental.pallas import tpu_sc as plsc


def _sc_gather(idx3, x, nw, nch, k):
    total = nw * nch * k
    _, c = x.shape
    mesh = plsc.VectorSubcoreMesh(core_axis_name="c", subcore_axis_name="s")
    nc = mesh.num_cores

    @functools.partial(
        pl.kernel,
        out_type=jax.ShapeDtypeStruct((total, c), jnp.float32),
        mesh=mesh,
        scratch_types=[
            pltpu.VMEM((nch, k), jnp.int32),
            pltpu.VMEM((k, c), jnp.float32),
            pltpu.VMEM((k, c), jnp.float32),
            pltpu.SemaphoreType.DMA,
            pltpu.SemaphoreType.DMA,
        ],
    )
    def gather_kernel(idx_hbm, x_hbm, out_hbm, idx_v, bufa, bufb, sema, semb):
        wid = lax.axis_index("s") * nc + lax.axis_index("c")
        base = wid * (nch * k)
        pltpu.sync_copy(idx_hbm.at[wid], idx_v)

        def pair(j, carry):
            c0 = j * 2
            c1 = c0 + 1
            cpa = pltpu.async_copy(x_hbm.at[idx_v.at[c0]], bufa, sema)
            cpb = pltpu.async_copy(x_hbm.at[idx_v.at[c1]], bufb, semb)
            cpa.wait()
            pltpu.sync_copy(bufa, out_hbm.at[pl.ds(base + c0 * k, k)])
            cpb.wait()
            pltpu.sync_copy(bufb, out_hbm.at[pl.ds(base + c1 * k, k)])
            return carry

        lax.fori_loop(0, nch // 2, pair, 0)

    return gather_kernel(idx3, x)


def _mm_stats_body(x_ref, g_ref, w_ref, y_ref, s_ref):
    i = pl.program_id(0)
    g0 = g_ref[0]
    g1 = g_ref[1]
    g2 = g_ref[2]
    g3 = g_ref[3]
    feat = jnp.concatenate(
        [
            x_ref[...],
            jnp.minimum(g0, g1),
            jnp.maximum(g0, g1),
            jnp.minimum(g2, g3),
            jnp.maximum(g2, g3),
        ],
        axis=1,
    ).astype(jnp.bfloat16)
    y = jnp.dot(feat, w_ref[...], preferred_element_type=jnp.float32)
    y_ref[...] = y.astype(jnp.bfloat16)
    srow = jnp.sum(y, axis=0)[None]
    qrow = jnp.sum(y * y, axis=0)[None]
    blk = jnp.concatenate(
        [srow, qrow, jnp.zeros((6, y.shape[1]), jnp.float32)], axis=0
    )

    @pl.when(i == 0)
    def _():
        s_ref[...] = blk

    @pl.when(i != 0)
    def _():
        s_ref[...] += blk


def _norm2_body(ya_ref, yb_ref, sa_ref, sb_ref, p_ref, o_ref, *, n_rows, na_blocks):
    i = pl.program_id(0)
    s0 = sa_ref[0] + sb_ref[0]
    s1 = sa_ref[1] + sb_ref[1]
    inv_n = 1.0 / n_rows
    mean = s0 * inv_n
    var = s1 * inv_n - mean * mean
    inv = lax.rsqrt(var + 1e-5)
    scale = p_ref[0] * inv
    shift = p_ref[1] - mean * scale
    y = jnp.where(
        i < na_blocks,
        ya_ref[...].astype(jnp.float32),
        yb_ref[...].astype(jnp.float32),
    )
    o_ref[...] = jnp.maximum(y * scale + shift, 0.0)


def _mm_call(x, g, wt, t, nblk, off, e_part, c, c_out):
    return pl.pallas_call(
        _mm_stats_body,
        grid=(nblk,),
        in_specs=[
            pl.BlockSpec((t, c), lambda i: (off + i, 0)),
            pl.BlockSpec((4, t, c), lambda i: (0, i, 0)),
            pl.BlockSpec((5 * c, c_out), lambda i: (0, 0)),
        ],
        out_specs=[
            pl.BlockSpec((t, c_out), lambda i: (i, 0)),
            pl.BlockSpec((8, c_out), lambda i: (0, 0)),
        ],
        out_shape=[
            jax.ShapeDtypeStruct((e_part, c_out), jnp.bfloat16),
            jax.ShapeDtypeStruct((8, c_out), jnp.float32),
        ],
    )(x, g, wt)


def kernel(x, nb, W, gamma, beta):
    e, c = x.shape
    c_out = W.shape[0]
    nw = 32

    idx = jnp.clip(nb.astype(jnp.int32), 0, e - 1)

    e0 = 102400
    e1 = e - e0
    t = 1600
    nb0 = e0 // t
    nb1 = e1 // t

    idx0 = idx[:e0].T.reshape(-1)
    idx1 = idx[e0:].T.reshape(-1)
    k0, nch0 = 128, (4 * e0) // (nw * 128)
    k1, nch1 = 120, (4 * e1) // (nw * 120)

    wt = W.T.astype(jnp.bfloat16)

    g0 = _sc_gather(idx0.reshape(nw, nch0, k0), x, nw, nch0, k0).reshape(
        4, e0, c
    )
    y0, st0 = _mm_call(x, g0, wt, t, nb0, 0, e0, c, c_out)

    g1 = _sc_gather(idx1.reshape(nw, nch1, k1), x, nw, nch1, k1).reshape(
        4, e1, c
    )
    y1, st1 = _mm_call(x, g1, wt, t, nb1, nb0, e1, c, c_out)

    params = jnp.concatenate(
        [gamma[None], beta[None], jnp.zeros((6, c_out), jnp.float32)], axis=0
    )

    out = pl.pallas_call(
        functools.partial(_norm2_body, n_rows=e, na_blocks=nb0),
        grid=(nb0 + nb1,),
        in_specs=[
            pl.BlockSpec((t, c_out), lambda i: (jnp.minimum(i, nb0 - 1), 0)),
            pl.BlockSpec((t, c_out), lambda i: (jnp.maximum(i - nb0, 0), 0)),
            pl.BlockSpec((8, c_out), lambda i: (0, 0)),
            pl.BlockSpec((8, c_out), lambda i: (0, 0)),
            pl.BlockSpec((8, c_out), lambda i: (0, 0)),
        ],
        out_specs=pl.BlockSpec((t, c_out), lambda i: (i, 0)),
        out_shape=jax.ShapeDtypeStruct((e, c_out), jnp.float32),
    )(y0, y1, st0, st1, params)

    return out


# speedup vs baseline: 5.5386x; 1.0260x over previous
import functools

import jax
import jax.numpy as jnp
from jax import lax
from jax.experimental import pallas as pl
from jax.experimental.pallas import tpu as pltpu
from jax.experimental.pallas import tpu_sc as plsc


def _sc_gather(idx3, x, nw, nch, k):
    total = nw * nch * k
    _, c = x.shape
    mesh = plsc.VectorSubcoreMesh(core_axis_name="c", subcore_axis_name="s")
    nc = mesh.num_cores

    @functools.partial(
        pl.kernel,
        out_type=jax.ShapeDtypeStruct((total, c), jnp.float32),
        mesh=mesh,
        scratch_types=[
            pltpu.VMEM((nch, k), jnp.int32),
            pltpu.VMEM((k, c), jnp.float32),
            pltpu.VMEM((k, c), jnp.float32),
            pltpu.SemaphoreType.DMA,
            pltpu.SemaphoreType.DMA,
        ],
    )
    def gather_kernel(idx_hbm, x_hbm, out_hbm, idx_v, bufa, bufb, sema, semb):
        wid = lax.axis_index("s") * nc + lax.axis_index("c")
        base = wid * (nch * k)
        pltpu.sync_copy(idx_hbm.at[wid], idx_v)

        def pair(j, carry):
            c0 = j * 2
            c1 = c0 + 1
            cpa = pltpu.async_copy(x_hbm.at[idx_v.at[c0]], bufa, sema)
            cpb = pltpu.async_copy(x_hbm.at[idx_v.at[c1]], bufb, semb)
            cpa.wait()
            pltpu.sync_copy(bufa, out_hbm.at[pl.ds(base + c0 * k, k)])
            cpb.wait()
            pltpu.sync_copy(bufb, out_hbm.at[pl.ds(base + c1 * k, k)])
            return carry

        lax.fori_loop(0, nch // 2, pair, 0)

    return gather_kernel(idx3, x)


def _mm_stats_body(x_ref, g_ref, w_ref, yin_ref, y_ref, s_ref):
    del yin_ref
    i = pl.program_id(0)
    g0 = g_ref[0]
    g1 = g_ref[1]
    g2 = g_ref[2]
    g3 = g_ref[3]
    feat = jnp.concatenate(
        [
            x_ref[...],
            jnp.minimum(g0, g1),
            jnp.maximum(g0, g1),
            jnp.minimum(g2, g3),
            jnp.maximum(g2, g3),
        ],
        axis=1,
    ).astype(jnp.bfloat16)
    y = jnp.dot(feat, w_ref[...], preferred_element_type=jnp.float32)
    y_ref[...] = y.astype(jnp.bfloat16)
    srow = jnp.sum(y, axis=0)[None]
    qrow = jnp.sum(y * y, axis=0)[None]
    blk = jnp.concatenate(
        [srow, qrow, jnp.zeros((6, y.shape[1]), jnp.float32)], axis=0
    )

    @pl.when(i == 0)
    def _():
        s_ref[...] = blk

    @pl.when(i != 0)
    def _():
        s_ref[...] += blk


def _norm2_body(y_ref, sa_ref, sb_ref, p_ref, o_ref, *, n_rows):
    s0 = sa_ref[0] + sb_ref[0]
    s1 = sa_ref[1] + sb_ref[1]
    inv_n = 1.0 / n_rows
    mean = s0 * inv_n
    var = s1 * inv_n - mean * mean
    inv = lax.rsqrt(var + 1e-5)
    scale = p_ref[0] * inv
    shift = p_ref[1] - mean * scale
    y = y_ref[...].astype(jnp.float32)
    o_ref[...] = jnp.maximum(y * scale + shift, 0.0)


def _mm_call(x, g, wt, y_in, t, nblk, off, c, c_out):
    e_full = y_in.shape[0]
    return pl.pallas_call(
        _mm_stats_body,
        grid=(nblk,),
        in_specs=[
            pl.BlockSpec((t, c), lambda i: (off + i, 0)),
            pl.BlockSpec((4, t, c), lambda i: (0, i, 0)),
            pl.BlockSpec((5 * c, c_out), lambda i: (0, 0)),
            pl.BlockSpec(memory_space=pltpu.MemorySpace.HBM),
        ],
        out_specs=[
            pl.BlockSpec((t, c_out), lambda i: (off + i, 0)),
            pl.BlockSpec((8, c_out), lambda i: (0, 0)),
        ],
        out_shape=[
            jax.ShapeDtypeStruct((e_full, c_out), jnp.bfloat16),
            jax.ShapeDtypeStruct((8, c_out), jnp.float32),
        ],
        input_output_aliases={3: 0},
    )(x, g, wt, y_in)


def kernel(x, nb, W, gamma, beta):
    e, c = x.shape
    c_out = W.shape[0]
    nw = 32

    idx = jnp.clip(nb.astype(jnp.int32), 0, e - 1)

    e0 = 102400
    e1 = e - e0
    t = 1600
    nb0 = e0 // t
    nb1 = e1 // t

    idx0 = idx[:e0].T.reshape(-1)
    idx1 = idx[e0:].T.reshape(-1)
    k0, nch0 = 128, (4 * e0) // (nw * 128)
    k1, nch1 = 120, (4 * e1) // (nw * 120)

    wt = W.T.astype(jnp.bfloat16)

    g0 = _sc_gather(idx0.reshape(nw, nch0, k0), x, nw, nch0, k0).reshape(
        4, e0, c
    )
    y_init = jnp.zeros((e, c_out), jnp.bfloat16)
    y0, st0 = _mm_call(x, g0, wt, y_init, t, nb0, 0, c, c_out)

    g1 = _sc_gather(idx1.reshape(nw, nch1, k1), x, nw, nch1, k1).reshape(
        4, e1, c
    )
    y_full, st1 = _mm_call(x, g1, wt, y0, t, nb1, nb0, c, c_out)

    params = jnp.concatenate(
        [gamma[None], beta[None], jnp.zeros((6, c_out), jnp.float32)], axis=0
    )

    t2 = 4000
    out = pl.pallas_call(
        functools.partial(_norm2_body, n_rows=e),
        grid=(e // t2,),
        in_specs=[
            pl.BlockSpec((t2, c_out), lambda i: (i, 0)),
            pl.BlockSpec((8, c_out), lambda i: (0, 0)),
            pl.BlockSpec((8, c_out), lambda i: (0, 0)),
            pl.BlockSpec((8, c_out), lambda i: (0, 0)),
        ],
        out_specs=pl.BlockSpec((t2, c_out), lambda i: (i, 0)),
        out_shape=jax.ShapeDtypeStruct((e, c_out), jnp.float32),
    )(y_full, st0, st1, params)

    return out


# speedup vs baseline: 5.6270x; 1.0160x over previous
import functools

import jax
import jax.numpy as jnp
from jax import lax
from jax.experimental import pallas as pl
from jax.experimental.pallas import tpu as pltpu
from jax.experimental.pallas import tpu_sc as plsc


def _sc_gather(idx3, x, nw, nch, k):
    total = nw * nch * k
    _, c = x.shape
    mesh = plsc.VectorSubcoreMesh(core_axis_name="c", subcore_axis_name="s")
    nc = mesh.num_cores

    @functools.partial(
        pl.kernel,
        out_type=jax.ShapeDtypeStruct((total, c), jnp.float32),
        mesh=mesh,
        scratch_types=[
            pltpu.VMEM((nch, k), jnp.int32),
            pltpu.VMEM((k, c), jnp.float32),
            pltpu.VMEM((k, c), jnp.float32),
            pltpu.SemaphoreType.DMA,
            pltpu.SemaphoreType.DMA,
        ],
    )
    def gather_kernel(idx_hbm, x_hbm, out_hbm, idx_v, bufa, bufb, sema, semb):
        wid = lax.axis_index("s") * nc + lax.axis_index("c")
        base = wid * (nch * k)
        pltpu.sync_copy(idx_hbm.at[wid], idx_v)

        def pair(j, carry):
            c0 = j * 2
            c1 = c0 + 1
            cpa = pltpu.async_copy(x_hbm.at[idx_v.at[c0]], bufa, sema)
            cpb = pltpu.async_copy(x_hbm.at[idx_v.at[c1]], bufb, semb)
            cpa.wait()
            pltpu.sync_copy(bufa, out_hbm.at[pl.ds(base + c0 * k, k)])
            cpb.wait()
            pltpu.sync_copy(bufb, out_hbm.at[pl.ds(base + c1 * k, k)])
            return carry

        lax.fori_loop(0, nch // 2, pair, 0)

    return gather_kernel(idx3, x)


def _mm_stats_body(x_ref, g_ref, w_ref, y_ref, s_ref):
    i = pl.program_id(0)
    g0 = g_ref[0]
    g1 = g_ref[1]
    g2 = g_ref[2]
    g3 = g_ref[3]
    feat = jnp.concatenate(
        [
            x_ref[...],
            jnp.minimum(g0, g1),
            jnp.maximum(g0, g1),
            jnp.minimum(g2, g3),
            jnp.maximum(g2, g3),
        ],
        axis=1,
    ).astype(jnp.bfloat16)
    y = jnp.dot(feat, w_ref[...], preferred_element_type=jnp.float32)
    y_ref[...] = y.astype(jnp.bfloat16)
    srow = jnp.sum(y, axis=0)[None]
    qrow = jnp.sum(y * y, axis=0)[None]
    blk = jnp.concatenate(
        [srow, qrow, jnp.zeros((6, y.shape[1]), jnp.float32)], axis=0
    )

    @pl.when(i == 0)
    def _():
        s_ref[...] = blk

    @pl.when(i != 0)
    def _():
        s_ref[...] += blk


def _norm2_body(y_ref, sa_ref, sb_ref, p_ref, o_ref, *, n_rows):
    s0 = sa_ref[0] + sb_ref[0]
    s1 = sa_ref[1] + sb_ref[1]
    inv_n = 1.0 / n_rows
    mean = s0 * inv_n
    var = s1 * inv_n - mean * mean
    inv = lax.rsqrt(var + 1e-5)
    scale = p_ref[0] * inv
    shift = p_ref[1] - mean * scale
    y = y_ref[...].astype(jnp.float32)
    o_ref[...] = jnp.maximum(y * scale + shift, 0.0)


def _mm_call(x, g, wt, y_in, t, nblk, off, c, c_out, e_full):
    in_specs = [
        pl.BlockSpec((t, c), lambda i: (off + i, 0)),
        pl.BlockSpec((4, t, c), lambda i: (0, i, 0)),
        pl.BlockSpec((5 * c, c_out), lambda i: (0, 0)),
    ]
    args = [x, g, wt]
    aliases = {}
    body = _mm_stats_body
    if y_in is not None:
        in_specs.append(pl.BlockSpec(memory_space=pltpu.MemorySpace.HBM))
        args.append(y_in)
        aliases = {3: 0}

        def body(x_ref, g_ref, w_ref, yin_ref, y_ref, s_ref):
            del yin_ref
            return _mm_stats_body(x_ref, g_ref, w_ref, y_ref, s_ref)

    return pl.pallas_call(
        body,
        grid=(nblk,),
        in_specs=in_specs,
        out_specs=[
            pl.BlockSpec((t, c_out), lambda i: (off + i, 0)),
            pl.BlockSpec((8, c_out), lambda i: (0, 0)),
        ],
        out_shape=[
            jax.ShapeDtypeStruct((e_full, c_out), jnp.bfloat16),
            jax.ShapeDtypeStruct((8, c_out), jnp.float32),
        ],
        input_output_aliases=aliases,
    )(*args)


def kernel(x, nb, W, gamma, beta):
    e, c = x.shape
    c_out = W.shape[0]
    nw = 32

    idx = jnp.clip(nb.astype(jnp.int32), 0, e - 1)

    e0 = 81920
    e1 = e - e0
    t = 1280
    nb0 = e0 // t
    nb1 = e1 // t

    idx0 = idx[:e0].T.reshape(-1)
    idx1 = idx[e0:].T.reshape(-1)
    k0, nch0 = 128, (4 * e0) // (nw * 128)
    k1, nch1 = 80, (4 * e1) // (nw * 80)

    wt = W.T.astype(jnp.bfloat16)

    g0 = _sc_gather(idx0.reshape(nw, nch0, k0), x, nw, nch0, k0).reshape(
        4, e0, c
    )
    y0, st0 = _mm_call(x, g0, wt, None, t, nb0, 0, c, c_out, e)

    g1 = _sc_gather(idx1.reshape(nw, nch1, k1), x, nw, nch1, k1).reshape(
        4, e1, c
    )
    y_full, st1 = _mm_call(x, g1, wt, y0, t, nb1, nb0, c, c_out, e)

    params = jnp.concatenate(
        [gamma[None], beta[None], jnp.zeros((6, c_out), jnp.float32)], axis=0
    )

    t2 = 4000
    out = pl.pallas_call(
        functools.partial(_norm2_body, n_rows=e),
        grid=(e // t2,),
        in_specs=[
            pl.BlockSpec((t2, c_out), lambda i: (i, 0)),
            pl.BlockSpec((8, c_out), lambda i: (0, 0)),
            pl.BlockSpec((8, c_out), lambda i: (0, 0)),
            pl.BlockSpec((8, c_out), lambda i: (0, 0)),
        ],
        out_specs=pl.BlockSpec((t2, c_out), lambda i: (i, 0)),
        out_shape=jax.ShapeDtypeStruct((e, c_out), jnp.float32),
    )(y_full, st0, st1, params)

    return out


# speedup vs baseline: 5.8303x; 1.0361x over previous
import functools

import jax
import jax.numpy as jnp
from jax import lax
from jax.experimental import pallas as pl
from jax.experimental.pallas import tpu as pltpu
from jax.experimental.pallas import tpu_sc as plsc


def _sc_gather(idx3, x, nw, nch, k):
    total = nw * nch * k
    _, c = x.shape
    mesh = plsc.VectorSubcoreMesh(core_axis_name="c", subcore_axis_name="s")
    nc = mesh.num_cores

    @functools.partial(
        pl.kernel,
        out_type=jax.ShapeDtypeStruct((total, c), jnp.float32),
        mesh=mesh,
        scratch_types=[
            pltpu.VMEM((nch, k), jnp.int32),
            pltpu.VMEM((k, c), jnp.float32),
            pltpu.VMEM((k, c), jnp.float32),
            pltpu.SemaphoreType.DMA,
            pltpu.SemaphoreType.DMA,
        ],
    )
    def gather_kernel(idx_hbm, x_hbm, out_hbm, idx_v, bufa, bufb, sema, semb):
        wid = lax.axis_index("s") * nc + lax.axis_index("c")
        base = wid * (nch * k)
        pltpu.sync_copy(idx_hbm.at[wid], idx_v)

        def pair(j, carry):
            c0 = j * 2
            c1 = c0 + 1
            cpa = pltpu.async_copy(x_hbm.at[idx_v.at[c0]], bufa, sema)
            cpb = pltpu.async_copy(x_hbm.at[idx_v.at[c1]], bufb, semb)
            cpa.wait()
            pltpu.sync_copy(bufa, out_hbm.at[pl.ds(base + c0 * k, k)])
            cpb.wait()
            pltpu.sync_copy(bufb, out_hbm.at[pl.ds(base + c1 * k, k)])
            return carry

        lax.fori_loop(0, nch // 2, pair, 0)

    return gather_kernel(idx3, x)


def _mm_stats_body(x_ref, g_ref, w_ref, y_ref, s_ref):
    i = pl.program_id(0)
    g0 = g_ref[0]
    g1 = g_ref[1]
    g2 = g_ref[2]
    g3 = g_ref[3]
    feat = jnp.concatenate(
        [
            x_ref[...],
            jnp.minimum(g0, g1),
            jnp.maximum(g0, g1),
            jnp.minimum(g2, g3),
            jnp.maximum(g2, g3),
        ],
        axis=1,
    ).astype(jnp.bfloat16)
    y = jnp.dot(feat, w_ref[...], preferred_element_type=jnp.float32)
    y_ref[...] = y.astype(jnp.bfloat16)
    srow = jnp.sum(y, axis=0)[None]
    qrow = jnp.sum(y * y, axis=0)[None]
    blk = jnp.concatenate(
        [srow, qrow, jnp.zeros((6, y.shape[1]), jnp.float32)], axis=0
    )

    @pl.when(i == 0)
    def _():
        s_ref[...] = blk

    @pl.when(i != 0)
    def _():
        s_ref[...] += blk


def _norm2_body(y_ref, st_ref, p_ref, o_ref, *, n_rows, n_parts):
    s0 = st_ref[0]
    s1 = st_ref[1]
    for p in range(1, n_parts):
        s0 = s0 + st_ref[8 * p]
        s1 = s1 + st_ref[8 * p + 1]
    inv_n = 1.0 / n_rows
    mean = s0 * inv_n
    var = s1 * inv_n - mean * mean
    inv = lax.rsqrt(var + 1e-5)
    scale = p_ref[0] * inv
    shift = p_ref[1] - mean * scale
    y = y_ref[...].astype(jnp.float32)
    o_ref[...] = jnp.maximum(y * scale + shift, 0.0)


def _mm_call(x, g, wt, y_in, t, nblk, off, c, c_out, e_full):
    in_specs = [
        pl.BlockSpec((t, c), lambda i: (off + i, 0)),
        pl.BlockSpec((4, t, c), lambda i: (0, i, 0)),
        pl.BlockSpec((5 * c, c_out), lambda i: (0, 0)),
    ]
    args = [x, g, wt]
    aliases = {}
    body = _mm_stats_body
    if y_in is not None:
        in_specs.append(pl.BlockSpec(memory_space=pltpu.MemorySpace.HBM))
        args.append(y_in)
        aliases = {3: 0}

        def body(x_ref, g_ref, w_ref, yin_ref, y_ref, s_ref):
            del yin_ref
            return _mm_stats_body(x_ref, g_ref, w_ref, y_ref, s_ref)

    return pl.pallas_call(
        body,
        grid=(nblk,),
        in_specs=in_specs,
        out_specs=[
            pl.BlockSpec((t, c_out), lambda i: (off + i, 0)),
            pl.BlockSpec((8, c_out), lambda i: (0, 0)),
        ],
        out_shape=[
            jax.ShapeDtypeStruct((e_full, c_out), jnp.bfloat16),
            jax.ShapeDtypeStruct((8, c_out), jnp.float32),
        ],
        input_output_aliases=aliases,
    )(*args)


def kernel(x, nb, W, gamma, beta):
    e, c = x.shape
    c_out = W.shape[0]
    nw = 32

    idx = jnp.clip(nb.astype(jnp.int32), 0, e - 1)

    t = 1280
    parts = [
        (0, 40960, 128, 40),
        (40960, 40960, 128, 40),
        (81920, 40960, 128, 40),
        (122880, 37120, 80, 58),
    ]

    wt = W.T.astype(jnp.bfloat16)

    y_cur = None
    stats = []
    for off, ecnt, kk, nch in parts:
        idx_p = idx[off : off + ecnt].T.reshape(nw, nch, kk)
        g_p = _sc_gather(idx_p, x, nw, nch, kk).reshape(4, ecnt, c)
        y_cur, st_p = _mm_call(
            x, g_p, wt, y_cur, t, ecnt // t, off // t, c, c_out, e
        )
        stats.append(st_p)

    st_all = jnp.concatenate(stats, axis=0)
    params = jnp.concatenate(
        [gamma[None], beta[None], jnp.zeros((6, c_out), jnp.float32)], axis=0
    )

    t2 = 4000
    out = pl.pallas_call(
        functools.partial(_norm2_body, n_rows=e, n_parts=len(parts)),
        grid=(e // t2,),
        in_specs=[
            pl.BlockSpec((t2, c_out), lambda i: (i, 0)),
            pl.BlockSpec((8 * len(parts), c_out), lambda i: (0, 0)),
            pl.BlockSpec((8, c_out), lambda i: (0, 0)),
        ],
        out_specs=pl.BlockSpec((t2, c_out), lambda i: (i, 0)),
        out_shape=jax.ShapeDtypeStruct((e, c_out), jnp.float32),
    )(y_cur, st_all, params)

    return out


# speedup vs baseline: 5.8923x; 1.0106x over previous
import functools

import jax
import jax.numpy as jnp
from jax import lax
from jax.experimental import pallas as pl
from jax.experimental.pallas import tpu as pltpu
from jax.experimental.pallas import tpu_sc as plsc


def _sc_gather(idx3, x, nw, nch, k):
    total = nw * nch * k
    _, c = x.shape
    mesh = plsc.VectorSubcoreMesh(core_axis_name="c", subcore_axis_name="s")
    nc = mesh.num_cores

    @functools.partial(
        pl.kernel,
        out_type=jax.ShapeDtypeStruct((total, c), jnp.float32),
        mesh=mesh,
        scratch_types=[
            pltpu.VMEM((nch, k), jnp.int32),
            pltpu.VMEM((k, c), jnp.float32),
            pltpu.VMEM((k, c), jnp.float32),
            pltpu.SemaphoreType.DMA,
            pltpu.SemaphoreType.DMA,
        ],
    )
    def gather_kernel(idx_hbm, x_hbm, out_hbm, idx_v, bufa, bufb, sema, semb):
        wid = lax.axis_index("s") * nc + lax.axis_index("c")
        base = wid * (nch * k)
        pltpu.sync_copy(idx_hbm.at[wid], idx_v)

        def pair(j, carry):
            c0 = j * 2
            c1 = c0 + 1
            cpa = pltpu.async_copy(x_hbm.at[idx_v.at[c0]], bufa, sema)
            cpb = pltpu.async_copy(x_hbm.at[idx_v.at[c1]], bufb, semb)
            cpa.wait()
            pltpu.sync_copy(bufa, out_hbm.at[pl.ds(base + c0 * k, k)])
            cpb.wait()
            pltpu.sync_copy(bufb, out_hbm.at[pl.ds(base + c1 * k, k)])
            return carry

        lax.fori_loop(0, nch // 2, pair, 0)

    return gather_kernel(idx3, x)


def _mm_stats_body(x_ref, g_ref, w_ref, y_ref, s_ref):
    i = pl.program_id(0)
    g0 = g_ref[0]
    g1 = g_ref[1]
    g2 = g_ref[2]
    g3 = g_ref[3]
    feat = jnp.concatenate(
        [
            x_ref[...],
            jnp.minimum(g0, g1),
            jnp.maximum(g0, g1),
            jnp.minimum(g2, g3),
            jnp.maximum(g2, g3),
        ],
        axis=1,
    ).astype(jnp.bfloat16)
    y = jnp.dot(feat, w_ref[...], preferred_element_type=jnp.float32)
    y_ref[...] = y.astype(jnp.bfloat16)
    srow = jnp.sum(y, axis=0)[None]
    qrow = jnp.sum(y * y, axis=0)[None]
    blk = jnp.concatenate(
        [srow, qrow, jnp.zeros((6, y.shape[1]), jnp.float32)], axis=0
    )

    @pl.when(i == 0)
    def _():
        s_ref[...] = blk

    @pl.when(i != 0)
    def _():
        s_ref[...] += blk


def _norm2_body(y_ref, st_ref, p_ref, o_ref, *, n_rows, n_parts):
    s0 = st_ref[0]
    s1 = st_ref[1]
    for p in range(1, n_parts):
        s0 = s0 + st_ref[8 * p]
        s1 = s1 + st_ref[8 * p + 1]
    inv_n = 1.0 / n_rows
    mean = s0 * inv_n
    var = s1 * inv_n - mean * mean
    inv = lax.rsqrt(var + 1e-5)
    scale = p_ref[0] * inv
    shift = p_ref[1] - mean * scale
    y = y_ref[...].astype(jnp.float32)
    o_ref[...] = jnp.maximum(y * scale + shift, 0.0)


def _mm_call(x, g, wt, y_in, t, nblk, off, c, c_out, e_full):
    in_specs = [
        pl.BlockSpec((t, c), lambda i: (off + i, 0)),
        pl.BlockSpec((4, t, c), lambda i: (0, i, 0)),
        pl.BlockSpec((5 * c, c_out), lambda i: (0, 0)),
    ]
    args = [x, g, wt]
    aliases = {}
    body = _mm_stats_body
    if y_in is not None:
        in_specs.append(pl.BlockSpec(memory_space=pltpu.MemorySpace.HBM))
        args.append(y_in)
        aliases = {3: 0}

        def body(x_ref, g_ref, w_ref, yin_ref, y_ref, s_ref):
            del yin_ref
            return _mm_stats_body(x_ref, g_ref, w_ref, y_ref, s_ref)

    return pl.pallas_call(
        body,
        grid=(nblk,),
        in_specs=in_specs,
        out_specs=[
            pl.BlockSpec((t, c_out), lambda i: (off + i, 0)),
            pl.BlockSpec((8, c_out), lambda i: (0, 0)),
        ],
        out_shape=[
            jax.ShapeDtypeStruct((e_full, c_out), jnp.bfloat16),
            jax.ShapeDtypeStruct((8, c_out), jnp.float32),
        ],
        input_output_aliases=aliases,
    )(*args)


def kernel(x, nb, W, gamma, beta):
    e, c = x.shape
    c_out = W.shape[0]
    nw = 32

    idx = jnp.clip(nb.astype(jnp.int32), 0, e - 1)

    t = 1280
    parts = [
        (0, 12800, 80, 20),
        (12800, 35840, 80, 56),
        (48640, 40960, 128, 40),
        (89600, 40960, 128, 40),
        (130560, 29440, 80, 46),
    ]

    wt = W.T.astype(jnp.bfloat16)

    y_cur = None
    stats = []
    for off, ecnt, kk, nch in parts:
        idx_p = idx[off : off + ecnt].T.reshape(nw, nch, kk)
        g_p = _sc_gather(idx_p, x, nw, nch, kk).reshape(4, ecnt, c)
        y_cur, st_p = _mm_call(
            x, g_p, wt, y_cur, t, ecnt // t, off // t, c, c_out, e
        )
        stats.append(st_p)

    st_all = jnp.concatenate(stats, axis=0)
    params = jnp.concatenate(
        [gamma[None], beta[None], jnp.zeros((6, c_out), jnp.float32)], axis=0
    )

    t2 = 8000
    out = pl.pallas_call(
        functools.partial(_norm2_body, n_rows=e, n_parts=len(parts)),
        grid=(e // t2,),
        in_specs=[
            pl.BlockSpec((t2, c_out), lambda i: (i, 0)),
            pl.BlockSpec((8 * len(parts), c_out), lambda i: (0, 0)),
            pl.BlockSpec((8, c_out), lambda i: (0, 0)),
        ],
        out_specs=pl.BlockSpec((t2, c_out), lambda i: (i, 0)),
        out_shape=jax.ShapeDtypeStruct((e, c_out), jnp.float32),
    )(y_cur, st_all, params)

    return out


# speedup vs baseline: 5.8976x; 1.0009x over previous
import functools

import jax
import jax.numpy as jnp
from jax import lax
from jax.experimental import pallas as pl
from jax.experimental.pallas import tpu as pltpu
from jax.experimental.pallas import tpu_sc as plsc


def _sc_gather(idx3, x, nw, nch, k):
    total = nw * nch * k
    _, c = x.shape
    mesh = plsc.VectorSubcoreMesh(core_axis_name="c", subcore_axis_name="s")
    nc = mesh.num_cores

    @functools.partial(
        pl.kernel,
        out_type=jax.ShapeDtypeStruct((total, c), jnp.float32),
        mesh=mesh,
        scratch_types=[
            pltpu.VMEM((nch, k), jnp.int32),
            pltpu.VMEM((k, c), jnp.float32),
            pltpu.VMEM((k, c), jnp.float32),
            pltpu.SemaphoreType.DMA,
            pltpu.SemaphoreType.DMA,
        ],
    )
    def gather_kernel(idx_hbm, x_hbm, out_hbm, idx_v, bufa, bufb, sema, semb):
        wid = lax.axis_index("s") * nc + lax.axis_index("c")
        base = wid * (nch * k)
        pltpu.sync_copy(idx_hbm.at[wid], idx_v)

        def pair(j, carry):
            c0 = j * 2
            c1 = c0 + 1
            cpa = pltpu.async_copy(x_hbm.at[idx_v.at[c0]], bufa, sema)
            cpb = pltpu.async_copy(x_hbm.at[idx_v.at[c1]], bufb, semb)
            cpa.wait()
            pltpu.sync_copy(bufa, out_hbm.at[pl.ds(base + c0 * k, k)])
            cpb.wait()
            pltpu.sync_copy(bufb, out_hbm.at[pl.ds(base + c1 * k, k)])
            return carry

        lax.fori_loop(0, nch // 2, pair, 0)

    return gather_kernel(idx3, x)


def _mm_stats_body(x_ref, g_ref, w_ref, y_ref, s_ref):
    i = pl.program_id(0)
    g0 = g_ref[0]
    g1 = g_ref[1]
    g2 = g_ref[2]
    g3 = g_ref[3]
    feat = jnp.concatenate(
        [
            x_ref[...],
            jnp.minimum(g0, g1),
            jnp.maximum(g0, g1),
            jnp.minimum(g2, g3),
            jnp.maximum(g2, g3),
        ],
        axis=1,
    ).astype(jnp.bfloat16)
    y = jnp.dot(feat, w_ref[...], preferred_element_type=jnp.float32)
    y_ref[...] = y.astype(jnp.bfloat16)
    srow = jnp.sum(y, axis=0)[None]
    qrow = jnp.sum(y * y, axis=0)[None]
    blk = jnp.concatenate(
        [srow, qrow, jnp.zeros((6, y.shape[1]), jnp.float32)], axis=0
    )

    @pl.when(i == 0)
    def _():
        s_ref[...] = blk

    @pl.when(i != 0)
    def _():
        s_ref[...] += blk


def _norm2_body(y_ref, st_ref, p_ref, o_ref, *, n_rows, n_parts):
    s0 = st_ref[0]
    s1 = st_ref[1]
    for p in range(1, n_parts):
        s0 = s0 + st_ref[8 * p]
        s1 = s1 + st_ref[8 * p + 1]
    inv_n = 1.0 / n_rows
    mean = s0 * inv_n
    var = s1 * inv_n - mean * mean
    inv = lax.rsqrt(var + 1e-5)
    scale = p_ref[0] * inv
    shift = p_ref[1] - mean * scale
    y = y_ref[...].astype(jnp.float32)
    o_ref[...] = jnp.maximum(y * scale + shift, 0.0)


def _mm_call(x, g, wt, y_in, t, nblk, off, c, c_out, e_full):
    in_specs = [
        pl.BlockSpec((t, c), lambda i: (off + i, 0)),
        pl.BlockSpec((4, t, c), lambda i: (0, i, 0)),
        pl.BlockSpec((5 * c, c_out), lambda i: (0, 0)),
    ]
    args = [x, g, wt]
    aliases = {}
    body = _mm_stats_body
    if y_in is not None:
        in_specs.append(pl.BlockSpec(memory_space=pltpu.MemorySpace.HBM))
        args.append(y_in)
        aliases = {3: 0}

        def body(x_ref, g_ref, w_ref, yin_ref, y_ref, s_ref):
            del yin_ref
            return _mm_stats_body(x_ref, g_ref, w_ref, y_ref, s_ref)

    return pl.pallas_call(
        body,
        grid=(nblk,),
        in_specs=in_specs,
        out_specs=[
            pl.BlockSpec((t, c_out), lambda i: (off + i, 0)),
            pl.BlockSpec((8, c_out), lambda i: (0, 0)),
        ],
        out_shape=[
            jax.ShapeDtypeStruct((e_full, c_out), jnp.bfloat16),
            jax.ShapeDtypeStruct((8, c_out), jnp.float32),
        ],
        input_output_aliases=aliases,
    )(*args)


def kernel(x, nb, W, gamma, beta):
    e, c = x.shape
    c_out = W.shape[0]
    nw = 32

    idx = jnp.clip(nb.astype(jnp.int32), 0, e - 1)

    parts = [
        (0, 20480, 128, 20, 2560),
        (20480, 40960, 128, 40, 2560),
        (61440, 51200, 128, 50, 2560),
        (112640, 47360, 80, 74, 1280),
    ]

    wt = W.T.astype(jnp.bfloat16)

    y_cur = None
    stats = []
    for off, ecnt, kk, nch, t in parts:
        idx_p = idx[off : off + ecnt].T.reshape(nw, nch, kk)
        g_p = _sc_gather(idx_p, x, nw, nch, kk).reshape(4, ecnt, c)
        y_cur, st_p = _mm_call(
            x, g_p, wt, y_cur, t, ecnt // t, off // t, c, c_out, e
        )
        stats.append(st_p)

    st_all = jnp.concatenate(stats, axis=0)
    params = jnp.concatenate(
        [gamma[None], beta[None], jnp.zeros((6, c_out), jnp.float32)], axis=0
    )

    t2 = 8000
    out = pl.pallas_call(
        functools.partial(_norm2_body, n_rows=e, n_parts=len(parts)),
        grid=(e // t2,),
        in_specs=[
            pl.BlockSpec((t2, c_out), lambda i: (i, 0)),
            pl.BlockSpec((8 * len(parts), c_out), lambda i: (0, 0)),
            pl.BlockSpec((8, c_out), lambda i: (0, 0)),
        ],
        out_specs=pl.BlockSpec((t2, c_out), lambda i: (i, 0)),
        out_shape=jax.ShapeDtypeStruct((e, c_out), jnp.float32),
    )(y_cur, st_all, params)

    return out


# speedup vs baseline: 6.0842x; 1.0316x over previous
import functools

import jax
import jax.numpy as jnp
from jax import lax
from jax.experimental import pallas as pl
from jax.experimental.pallas import tpu as pltpu
from jax.experimental.pallas import tpu_sc as plsc


def _sc_gather(idx3, x, nw, nch, k):
    total = nw * nch * k
    _, c = x.shape
    mesh = plsc.VectorSubcoreMesh(core_axis_name="c", subcore_axis_name="s")
    nc = mesh.num_cores

    nbuf = 4

    @functools.partial(
        pl.kernel,
        out_type=jax.ShapeDtypeStruct((total, c), jnp.float32),
        mesh=mesh,
        scratch_types=[
            pltpu.VMEM((nch, k), jnp.int32),
            [pltpu.VMEM((k, c), jnp.float32) for _ in range(nbuf)],
            [pltpu.SemaphoreType.DMA for _ in range(nbuf)],
            [pltpu.SemaphoreType.DMA for _ in range(nbuf)],
        ],
    )
    def gather_kernel(idx_hbm, x_hbm, out_hbm, idx_v, bufs, gsems, wsems):
        wid = lax.axis_index("s") * nc + lax.axis_index("c")
        base = wid * (nch * k)
        pltpu.sync_copy(idx_hbm.at[wid], idx_v)

        for b in range(nbuf):
            if b < nch:
                pltpu.async_copy(x_hbm.at[idx_v.at[b]], bufs[b], gsems[b])

        def quad(j, carry):
            c0 = j * nbuf
            for b in range(nbuf):
                cc = c0 + b

                @pl.when(cc < nch)
                def _(b=b, cc=cc):
                    pltpu.make_async_copy(
                        x_hbm.at[idx_v.at[cc]], bufs[b], gsems[b]
                    ).wait()
                    pltpu.async_copy(
                        bufs[b], out_hbm.at[pl.ds(base + cc * k, k)], wsems[b]
                    )

            for b in range(nbuf):
                cc = c0 + b

                @pl.when(cc < nch)
                def _(b=b, cc=cc):
                    pltpu.make_async_copy(
                        bufs[b], out_hbm.at[pl.ds(base + cc * k, k)], wsems[b]
                    ).wait()

                @pl.when(cc + nbuf < nch)
                def _(b=b, cc=cc):
                    pltpu.async_copy(
                        x_hbm.at[idx_v.at[cc + nbuf]], bufs[b], gsems[b]
                    )

            return carry

        lax.fori_loop(0, (nch + nbuf - 1) // nbuf, quad, 0)

    return gather_kernel(idx3, x)


def _mm_stats_body(x_ref, g_ref, w_ref, y_ref, s_ref):
    i = pl.program_id(0)
    g0 = g_ref[0]
    g1 = g_ref[1]
    g2 = g_ref[2]
    g3 = g_ref[3]
    feat = jnp.concatenate(
        [
            x_ref[...],
            jnp.minimum(g0, g1),
            jnp.maximum(g0, g1),
            jnp.minimum(g2, g3),
            jnp.maximum(g2, g3),
        ],
        axis=1,
    ).astype(jnp.bfloat16)
    y = jnp.dot(feat, w_ref[...], preferred_element_type=jnp.float32)
    y_ref[...] = y.astype(jnp.bfloat16)
    srow = jnp.sum(y, axis=0)[None]
    qrow = jnp.sum(y * y, axis=0)[None]
    blk = jnp.concatenate(
        [srow, qrow, jnp.zeros((6, y.shape[1]), jnp.float32)], axis=0
    )

    @pl.when(i == 0)
    def _():
        s_ref[...] = blk

    @pl.when(i != 0)
    def _():
        s_ref[...] += blk


def _norm2_body(y_ref, st_ref, p_ref, o_ref, *, n_rows, n_parts):
    s0 = st_ref[0]
    s1 = st_ref[1]
    for p in range(1, n_parts):
        s0 = s0 + st_ref[8 * p]
        s1 = s1 + st_ref[8 * p + 1]
    inv_n = 1.0 / n_rows
    mean = s0 * inv_n
    var = s1 * inv_n - mean * mean
    inv = lax.rsqrt(var + 1e-5)
    scale = p_ref[0] * inv
    shift = p_ref[1] - mean * scale
    y = y_ref[...].astype(jnp.float32)
    o_ref[...] = jnp.maximum(y * scale + shift, 0.0)


def _mm_call(x, g, wt, y_in, t, nblk, off, c, c_out, e_full):
    in_specs = [
        pl.BlockSpec((t, c), lambda i: (off + i, 0)),
        pl.BlockSpec((4, t, c), lambda i: (0, i, 0)),
        pl.BlockSpec((5 * c, c_out), lambda i: (0, 0)),
    ]
    args = [x, g, wt]
    aliases = {}
    body = _mm_stats_body
    if y_in is not None:
        in_specs.append(pl.BlockSpec(memory_space=pltpu.MemorySpace.HBM))
        args.append(y_in)
        aliases = {3: 0}

        def body(x_ref, g_ref, w_ref, yin_ref, y_ref, s_ref):
            del yin_ref
            return _mm_stats_body(x_ref, g_ref, w_ref, y_ref, s_ref)

    return pl.pallas_call(
        body,
        grid=(nblk,),
        in_specs=in_specs,
        out_specs=[
            pl.BlockSpec((t, c_out), lambda i: (off + i, 0)),
            pl.BlockSpec((8, c_out), lambda i: (0, 0)),
        ],
        out_shape=[
            jax.ShapeDtypeStruct((e_full, c_out), jnp.bfloat16),
            jax.ShapeDtypeStruct((8, c_out), jnp.float32),
        ],
        input_output_aliases=aliases,
    )(*args)


def kernel(x, nb, W, gamma, beta):
    e, c = x.shape
    c_out = W.shape[0]
    nw = 32

    idx = jnp.clip(nb.astype(jnp.int32), 0, e - 1)

    parts = [
        (0, 20480, 128, 20, 2560),
        (20480, 40960, 128, 40, 2560),
        (61440, 51200, 128, 50, 2560),
        (112640, 47360, 80, 74, 1280),
    ]

    wt = W.T.astype(jnp.bfloat16)

    y_cur = None
    stats = []
    for off, ecnt, kk, nch, t in parts:
        idx_p = idx[off : off + ecnt].T.reshape(nw, nch, kk)
        g_p = _sc_gather(idx_p, x, nw, nch, kk).reshape(4, ecnt, c)
        y_cur, st_p = _mm_call(
            x, g_p, wt, y_cur, t, ecnt // t, off // t, c, c_out, e
        )
        stats.append(st_p)

    st_all = jnp.concatenate(stats, axis=0)
    params = jnp.concatenate(
        [gamma[None], beta[None], jnp.zeros((6, c_out), jnp.float32)], axis=0
    )

    t2 = 8000
    out = pl.pallas_call(
        functools.partial(_norm2_body, n_rows=e, n_parts=len(parts)),
        grid=(e // t2,),
        in_specs=[
            pl.BlockSpec((t2, c_out), lambda i: (i, 0)),
            pl.BlockSpec((8 * len(parts), c_out), lambda i: (0, 0)),
            pl.BlockSpec((8, c_out), lambda i: (0, 0)),
        ],
        out_specs=pl.BlockSpec((t2, c_out), lambda i: (i, 0)),
        out_shape=jax.ShapeDtypeStruct((e, c_out), jnp.float32),
    )(y_cur, st_all, params)

    return out


# speedup vs baseline: 6.1329x; 1.0080x over previous
import functools

import jax
import jax.numpy as jnp
from jax import lax
from jax.experimental import pallas as pl
from jax.experimental.pallas import tpu as pltpu
from jax.experimental.pallas import tpu_sc as plsc


def _sc_gather(idx3, x, nw, nch, k):
    total = nw * nch * k
    _, c = x.shape
    mesh = plsc.VectorSubcoreMesh(core_axis_name="c", subcore_axis_name="s")
    nc = mesh.num_cores

    nbuf = 6

    @functools.partial(
        pl.kernel,
        out_type=jax.ShapeDtypeStruct((total, c), jnp.float32),
        mesh=mesh,
        scratch_types=[
            pltpu.VMEM((nch, k), jnp.int32),
            [pltpu.VMEM((k, c), jnp.float32) for _ in range(nbuf)],
            [pltpu.SemaphoreType.DMA for _ in range(nbuf)],
            [pltpu.SemaphoreType.DMA for _ in range(nbuf)],
        ],
    )
    def gather_kernel(idx_hbm, x_hbm, out_hbm, idx_v, bufs, gsems, wsems):
        wid = lax.axis_index("s") * nc + lax.axis_index("c")
        base = wid * (nch * k)
        pltpu.sync_copy(idx_hbm.at[wid], idx_v)

        for b in range(nbuf):
            if b < nch:
                pltpu.async_copy(x_hbm.at[idx_v.at[b]], bufs[b], gsems[b])

        def quad(j, carry):
            c0 = j * nbuf
            for b in range(nbuf):
                cc = c0 + b

                @pl.when(cc < nch)
                def _(b=b, cc=cc):
                    pltpu.make_async_copy(
                        x_hbm.at[idx_v.at[cc]], bufs[b], gsems[b]
                    ).wait()
                    pltpu.async_copy(
                        bufs[b], out_hbm.at[pl.ds(base + cc * k, k)], wsems[b]
                    )

            for b in range(nbuf):
                cc = c0 + b

                @pl.when(cc < nch)
                def _(b=b, cc=cc):
                    pltpu.make_async_copy(
                        bufs[b], out_hbm.at[pl.ds(base + cc * k, k)], wsems[b]
                    ).wait()

                @pl.when(cc + nbuf < nch)
                def _(b=b, cc=cc):
                    pltpu.async_copy(
                        x_hbm.at[idx_v.at[cc + nbuf]], bufs[b], gsems[b]
                    )

            return carry

        lax.fori_loop(0, (nch + nbuf - 1) // nbuf, quad, 0)

    return gather_kernel(idx3, x)


def _mm_stats_body(x_ref, g_ref, w_ref, y_ref, s_ref):
    i = pl.program_id(0)
    g0 = g_ref[0]
    g1 = g_ref[1]
    g2 = g_ref[2]
    g3 = g_ref[3]
    feat = jnp.concatenate(
        [
            x_ref[...],
            jnp.minimum(g0, g1),
            jnp.maximum(g0, g1),
            jnp.minimum(g2, g3),
            jnp.maximum(g2, g3),
        ],
        axis=1,
    ).astype(jnp.bfloat16)
    y = jnp.dot(feat, w_ref[...], preferred_element_type=jnp.float32)
    y_ref[...] = y.astype(jnp.bfloat16)
    srow = jnp.sum(y, axis=0)[None]
    qrow = jnp.sum(y * y, axis=0)[None]
    blk = jnp.concatenate(
        [srow, qrow, jnp.zeros((6, y.shape[1]), jnp.float32)], axis=0
    )

    @pl.when(i == 0)
    def _():
        s_ref[...] = blk

    @pl.when(i != 0)
    def _():
        s_ref[...] += blk


def _norm2_body(y_ref, st_ref, p_ref, o_ref, *, n_rows, n_parts):
    s0 = st_ref[0]
    s1 = st_ref[1]
    for p in range(1, n_parts):
        s0 = s0 + st_ref[8 * p]
        s1 = s1 + st_ref[8 * p + 1]
    inv_n = 1.0 / n_rows
    mean = s0 * inv_n
    var = s1 * inv_n - mean * mean
    inv = lax.rsqrt(var + 1e-5)
    scale = p_ref[0] * inv
    shift = p_ref[1] - mean * scale
    y = y_ref[...].astype(jnp.float32)
    o_ref[...] = jnp.maximum(y * scale + shift, 0.0)


def _mm_call(x, g, wt, y_in, t, nblk, off, c, c_out, e_full):
    in_specs = [
        pl.BlockSpec((t, c), lambda i: (off + i, 0)),
        pl.BlockSpec((4, t, c), lambda i: (0, i, 0)),
        pl.BlockSpec((5 * c, c_out), lambda i: (0, 0)),
    ]
    args = [x, g, wt]
    aliases = {}
    body = _mm_stats_body
    if y_in is not None:
        in_specs.append(pl.BlockSpec(memory_space=pltpu.MemorySpace.HBM))
        args.append(y_in)
        aliases = {3: 0}

        def body(x_ref, g_ref, w_ref, yin_ref, y_ref, s_ref):
            del yin_ref
            return _mm_stats_body(x_ref, g_ref, w_ref, y_ref, s_ref)

    return pl.pallas_call(
        body,
        grid=(nblk,),
        in_specs=in_specs,
        out_specs=[
            pl.BlockSpec((t, c_out), lambda i: (off + i, 0)),
            pl.BlockSpec((8, c_out), lambda i: (0, 0)),
        ],
        out_shape=[
            jax.ShapeDtypeStruct((e_full, c_out), jnp.bfloat16),
            jax.ShapeDtypeStruct((8, c_out), jnp.float32),
        ],
        input_output_aliases=aliases,
    )(*args)


def kernel(x, nb, W, gamma, beta):
    e, c = x.shape
    c_out = W.shape[0]
    nw = 32

    idx = jnp.clip(nb.astype(jnp.int32), 0, e - 1)

    parts = [
        (0, 20480, 128, 20, 2560),
        (20480, 40960, 128, 40, 2560),
        (61440, 51200, 128, 50, 2560),
        (112640, 47360, 80, 74, 1280),
    ]

    wt = W.T.astype(jnp.bfloat16)

    y_cur = None
    stats = []
    for off, ecnt, kk, nch, t in parts:
        idx_p = idx[off : off + ecnt].T.reshape(nw, nch, kk)
        g_p = _sc_gather(idx_p, x, nw, nch, kk).reshape(4, ecnt, c)
        y_cur, st_p = _mm_call(
            x, g_p, wt, y_cur, t, ecnt // t, off // t, c, c_out, e
        )
        stats.append(st_p)

    st_all = jnp.concatenate(stats, axis=0)
    params = jnp.concatenate(
        [gamma[None], beta[None], jnp.zeros((6, c_out), jnp.float32)], axis=0
    )

    t2 = 8000
    out = pl.pallas_call(
        functools.partial(_norm2_body, n_rows=e, n_parts=len(parts)),
        grid=(e // t2,),
        in_specs=[
            pl.BlockSpec((t2, c_out), lambda i: (i, 0)),
            pl.BlockSpec((8 * len(parts), c_out), lambda i: (0, 0)),
            pl.BlockSpec((8, c_out), lambda i: (0, 0)),
        ],
        out_specs=pl.BlockSpec((t2, c_out), lambda i: (i, 0)),
        out_shape=jax.ShapeDtypeStruct((e, c_out), jnp.float32),
    )(y_cur, st_all, params)

    return out
